# initial kernel scaffold (unmeasured)
import jax
import jax.numpy as jnp
from jax import lax
from jax.experimental import pallas as pl
from jax.experimental.pallas import tpu as pltpu

N_DEV = 4
SQ = 256
SKV = 4096
HQ = 8
DH = 128
D = HQ * DH
SCALE = 0.08838834764831843


def kernel(x, Wq, K_ext, V_ext, Wo):
    def body(x_ref, wq_ref, k_ref, v_ref, wo_ref, out_ref,
             comm_o, comm_ml, send_sems, recv_sems):
        my = lax.axis_index("i")
        left = lax.rem(my + N_DEV - 1, N_DEV)
        right = lax.rem(my + 1, N_DEV)

        barrier = pltpu.get_barrier_semaphore()
        for nbr in (left, right):
            pl.semaphore_signal(
                barrier, inc=1,
                device_id=(nbr,), device_id_type=pl.DeviceIdType.MESH,
            )
        pl.semaphore_wait(barrier, 2)

        xb = x_ref[0].astype(jnp.bfloat16)
        q = jnp.dot(xb, wq_ref[...].astype(jnp.bfloat16),
                    preferred_element_type=jnp.float32)

        qb = lax.broadcasted_iota(jnp.int32, (SQ, SKV), 0) // 64
        kbm = (lax.broadcasted_iota(jnp.int32, (SQ, SKV), 1) // 64) % 4
        mask = qb == kbm

        for h in range(HQ):
            qh = q[:, h * DH:(h + 1) * DH].astype(jnp.bfloat16)
            kh = k_ref[0, :, h, :].astype(jnp.bfloat16)
            s = lax.dot_general(
                qh, kh, (((1,), (1,)), ((), ())),
                preferred_element_type=jnp.float32) * SCALE
            s = jnp.where(mask, s, -1e9)
            m = jnp.max(s, axis=1)
            w = jnp.exp(s - m[:, None])
            l = jnp.sum(w, axis=1)
            vh = v_ref[0, :, h, :].astype(jnp.bfloat16)
            o = jnp.dot(w.astype(jnp.bfloat16), vh,
                        preferred_element_type=jnp.float32)
            comm_o[0, h * SQ:(h + 1) * SQ, :] = o
            comm_ml[0, h, :] = m
            comm_ml[0, HQ + h, :] = l

        for hop in range(N_DEV - 1):
            rdma_o = pltpu.make_async_remote_copy(
                src_ref=comm_o.at[hop],
                dst_ref=comm_o.at[hop + 1],
                send_sem=send_sems.at[hop, 0],
                recv_sem=recv_sems.at[hop, 0],
                device_id=(right,),
                device_id_type=pl.DeviceIdType.MESH,
            )
            rdma_ml = pltpu.make_async_remote_copy(
                src_ref=comm_ml.at[hop],
                dst_ref=comm_ml.at[hop + 1],
                send_sem=send_sems.at[hop, 1],
                recv_sem=recv_sems.at[hop, 1],
                device_id=(right,),
                device_id_type=pl.DeviceIdType.MESH,
            )
            rdma_o.start()
            rdma_ml.start()
            rdma_o.wait()
            rdma_ml.wait()

        ctx_heads = []
        for h in range(HQ):
            ms = [comm_ml[sl, h, :] for sl in range(N_DEV)]
            gm = jnp.maximum(jnp.maximum(ms[0], ms[1]),
                             jnp.maximum(ms[2], ms[3]))
            num = jnp.zeros((SQ, DH), jnp.float32)
            den = jnp.zeros((SQ,), jnp.float32)
            for sl in range(N_DEV):
                sc = jnp.exp(ms[sl] - gm)
                den = den + sc * comm_ml[sl, HQ + h, :]
                num = num + sc[:, None] * comm_o[sl, h * SQ:(h + 1) * SQ, :]
            ctx_heads.append(num / den[:, None])
        ctx = jnp.concatenate(ctx_heads, axis=1)

        out = jnp.dot(ctx.astype(jnp.bfloat16),
                      wo_ref[...].astype(jnp.bfloat16),
                      preferred_element_type=jnp.float32)
        out_ref[0] = out

    return pl.pallas_call(
        body,
        out_shape=jax.ShapeDtypeStruct((1, SQ, D), jnp.float32),
        in_specs=[pl.BlockSpec(memory_space=pltpu.VMEM)] * 5,
        out_specs=pl.BlockSpec(memory_space=pltpu.VMEM),
        scratch_shapes=[
            pltpu.VMEM((N_DEV, HQ * SQ, DH), jnp.float32),
            pltpu.VMEM((N_DEV, 2 * HQ, SQ), jnp.float32),
            pltpu.SemaphoreType.DMA((N_DEV - 1, 2)),
            pltpu.SemaphoreType.DMA((N_DEV - 1, 2)),
        ],
        compiler_params=pltpu.CompilerParams(collective_id=0),
    )(x, Wq, K_ext, V_ext, Wo)


# baseline (device time: 118283 ns/iter reference)
import jax
import jax.numpy as jnp
from jax import lax
from jax.experimental import pallas as pl
from jax.experimental.pallas import tpu as pltpu

N_DEV = 4
SQ = 256
SKV = 4096
HQ = 8
DH = 128
D = HQ * DH
SCALE = 0.08838834764831843


def kernel(x, Wq, K_ext, V_ext, Wo):
    def body(x_ref, wq_ref, k_ref, v_ref, wo_ref, out_ref,
             comm_o, comm_ml, send_sems, recv_sems):
        my = lax.axis_index("i")
        left = lax.rem(my + N_DEV - 1, N_DEV)
        right = lax.rem(my + 1, N_DEV)

        barrier = pltpu.get_barrier_semaphore()
        for nbr in (left, right):
            pl.semaphore_signal(
                barrier, inc=1,
                device_id=(nbr,), device_id_type=pl.DeviceIdType.MESH,
            )
        pl.semaphore_wait(barrier, 2)

        xb = x_ref[0].astype(jnp.bfloat16)
        q = jnp.dot(xb, wq_ref[...].astype(jnp.bfloat16),
                    preferred_element_type=jnp.float32)

        qb = lax.broadcasted_iota(jnp.int32, (SQ, SKV), 0) // 64
        kbm = (lax.broadcasted_iota(jnp.int32, (SQ, SKV), 1) // 64) % 4
        mask = qb == kbm

        for h in range(HQ):
            qh = q[:, h * DH:(h + 1) * DH].astype(jnp.bfloat16)
            kh = k_ref[0, :, h, :].astype(jnp.bfloat16)
            s = lax.dot_general(
                qh, kh, (((1,), (1,)), ((), ())),
                preferred_element_type=jnp.float32) * SCALE
            s = jnp.where(mask, s, -1e9)
            m = jnp.max(s, axis=1)
            w = jnp.exp(s - m[:, None])
            l = jnp.sum(w, axis=1)
            vh = v_ref[0, :, h, :].astype(jnp.bfloat16)
            o = jnp.dot(w.astype(jnp.bfloat16), vh,
                        preferred_element_type=jnp.float32)
            comm_o[0, h * SQ:(h + 1) * SQ, :] = o
            comm_ml[0, h, :] = m
            comm_ml[0, HQ + h, :] = l

        for hop in range(N_DEV - 1):
            rdma_o = pltpu.make_async_remote_copy(
                src_ref=comm_o.at[hop],
                dst_ref=comm_o.at[hop + 1],
                send_sem=send_sems.at[hop, 0],
                recv_sem=recv_sems.at[hop, 0],
                device_id=(right,),
                device_id_type=pl.DeviceIdType.MESH,
            )
            rdma_ml = pltpu.make_async_remote_copy(
                src_ref=comm_ml.at[hop],
                dst_ref=comm_ml.at[hop + 1],
                send_sem=send_sems.at[hop, 1],
                recv_sem=recv_sems.at[hop, 1],
                device_id=(right,),
                device_id_type=pl.DeviceIdType.MESH,
            )
            rdma_o.start()
            rdma_ml.start()
            rdma_o.wait()
            rdma_ml.wait()

        ctx_heads = []
        for h in range(HQ):
            ms = [comm_ml[sl, h, :] for sl in range(N_DEV)]
            gm = jnp.maximum(jnp.maximum(ms[0], ms[1]),
                             jnp.maximum(ms[2], ms[3]))
            num = jnp.zeros((SQ, DH), jnp.float32)
            den = jnp.zeros((SQ,), jnp.float32)
            for sl in range(N_DEV):
                sc = jnp.exp(ms[sl] - gm)
                den = den + sc * comm_ml[sl, HQ + h, :]
                num = num + sc[:, None] * comm_o[sl, h * SQ:(h + 1) * SQ, :]
            ctx_heads.append(num / den[:, None])
        ctx = jnp.concatenate(ctx_heads, axis=1)

        out = jnp.dot(ctx.astype(jnp.bfloat16),
                      wo_ref[...].astype(jnp.bfloat16),
                      preferred_element_type=jnp.float32)
        out_ref[0] = out

    return pl.pallas_call(
        body,
        out_shape=jax.ShapeDtypeStruct((1, SQ, D), jnp.float32),
        in_specs=[pl.BlockSpec(memory_space=pltpu.VMEM)] * 5,
        out_specs=pl.BlockSpec(memory_space=pltpu.VMEM),
        scratch_shapes=[
            pltpu.VMEM((N_DEV, HQ * SQ, DH), jnp.float32),
            pltpu.VMEM((N_DEV, 2 * HQ, SQ), jnp.float32),
            pltpu.SemaphoreType.DMA((N_DEV - 1, 2)),
            pltpu.SemaphoreType.DMA((N_DEV - 1, 2)),
        ],
        compiler_params=pltpu.CompilerParams(
            collective_id=0, vmem_limit_bytes=100 * 1024 * 1024),
    )(x, Wq, K_ext, V_ext, Wo)


# device time: 76147 ns/iter; 1.5534x vs baseline; 1.5534x over previous
import jax
import jax.numpy as jnp
from jax import lax
from jax.experimental import pallas as pl
from jax.experimental.pallas import tpu as pltpu

N_DEV = 4
SQ = 256
SKV = 4096
HQ = 8
DH = 128
D = HQ * DH
QB = 64
NSEL = SKV // 4
SCALE = 0.08838834764831843


def _sel(a, qb):
    return jnp.concatenate(
        [a[(qb + 4 * t) * 64:(qb + 4 * t + 1) * 64, :] for t in range(16)],
        axis=0)


def kernel(x, Wq, K_ext, V_ext, Wo):
    def body(x_ref, wq_ref, k_ref, v_ref, wo_ref, out_ref,
             comm_o, comm_ml, send_sems, recv_sems):
        my = lax.axis_index("i")
        peers = [lax.rem(my + d, N_DEV) for d in range(1, N_DEV)]

        barrier = pltpu.get_barrier_semaphore()
        for p in peers:
            pl.semaphore_signal(
                barrier, inc=1,
                device_id=(p,), device_id_type=pl.DeviceIdType.MESH,
            )
        pl.semaphore_wait(barrier, N_DEV - 1)

        xb = x_ref[0].astype(jnp.bfloat16)
        q = jnp.dot(xb, wq_ref[...].astype(jnp.bfloat16),
                    preferred_element_type=jnp.float32)

        for h in range(HQ):
            kh = k_ref[0, :, h, :].astype(jnp.bfloat16)
            vh = v_ref[0, :, h, :].astype(jnp.bfloat16)
            for qb in range(4):
                qhb = q[qb * QB:(qb + 1) * QB,
                        h * DH:(h + 1) * DH].astype(jnp.bfloat16)
                ksel = _sel(kh, qb)
                s = lax.dot_general(
                    qhb, ksel, (((1,), (1,)), ((), ())),
                    preferred_element_type=jnp.float32) * SCALE
                m = jnp.max(s, axis=1)
                w = jnp.exp(s - m[:, None])
                l = jnp.sum(w, axis=1)
                o = jnp.dot(w.astype(jnp.bfloat16), _sel(vh, qb),
                            preferred_element_type=jnp.float32)
                r0 = h * SQ + qb * QB
                comm_o[my, r0:r0 + QB, :] = o.astype(jnp.bfloat16)
                comm_ml[my, h, qb * QB:(qb + 1) * QB] = m
                comm_ml[my, HQ + h, qb * QB:(qb + 1) * QB] = l

        sends = []
        for d, p in enumerate(peers):
            rdma_o = pltpu.make_async_remote_copy(
                src_ref=comm_o.at[my],
                dst_ref=comm_o.at[my],
                send_sem=send_sems.at[d, 0],
                recv_sem=recv_sems.at[my, 0],
                device_id=(p,),
                device_id_type=pl.DeviceIdType.MESH,
            )
            rdma_ml = pltpu.make_async_remote_copy(
                src_ref=comm_ml.at[my],
                dst_ref=comm_ml.at[my],
                send_sem=send_sems.at[d, 1],
                recv_sem=recv_sems.at[my, 1],
                device_id=(p,),
                device_id_type=pl.DeviceIdType.MESH,
            )
            rdma_o.start()
            rdma_ml.start()
            sends.append((rdma_o, rdma_ml))

        for p in peers:
            for j in range(2):
                ref = comm_o if j == 0 else comm_ml
                pltpu.make_async_remote_copy(
                    src_ref=ref.at[my],
                    dst_ref=ref.at[p],
                    send_sem=send_sems.at[0, j],
                    recv_sem=recv_sems.at[p, j],
                    device_id=(p,),
                    device_id_type=pl.DeviceIdType.MESH,
                ).wait_recv()

        ctx_heads = []
        for h in range(HQ):
            ms = [comm_ml[sl, h, :] for sl in range(N_DEV)]
            gm = jnp.maximum(jnp.maximum(ms[0], ms[1]),
                             jnp.maximum(ms[2], ms[3]))
            num = jnp.zeros((SQ, DH), jnp.float32)
            den = jnp.zeros((SQ,), jnp.float32)
            for sl in range(N_DEV):
                sc = jnp.exp(ms[sl] - gm)
                den = den + sc * comm_ml[sl, HQ + h, :]
                num = num + sc[:, None] * comm_o[sl, h * SQ:(h + 1) * SQ,
                                                 :].astype(jnp.float32)
            ctx_heads.append(num / den[:, None])
        ctx = jnp.concatenate(ctx_heads, axis=1)

        out = jnp.dot(ctx.astype(jnp.bfloat16),
                      wo_ref[...].astype(jnp.bfloat16),
                      preferred_element_type=jnp.float32)
        out_ref[0] = out

        for rdma_o, rdma_ml in sends:
            rdma_o.wait_send()
            rdma_ml.wait_send()

    return pl.pallas_call(
        body,
        out_shape=jax.ShapeDtypeStruct((1, SQ, D), jnp.float32),
        in_specs=[pl.BlockSpec(memory_space=pltpu.VMEM)] * 5,
        out_specs=pl.BlockSpec(memory_space=pltpu.VMEM),
        scratch_shapes=[
            pltpu.VMEM((N_DEV, HQ * SQ, DH), jnp.bfloat16),
            pltpu.VMEM((N_DEV, 2 * HQ, SQ), jnp.float32),
            pltpu.SemaphoreType.DMA((N_DEV - 1, 2)),
            pltpu.SemaphoreType.DMA((N_DEV, 2)),
        ],
        compiler_params=pltpu.CompilerParams(
            collective_id=0, vmem_limit_bytes=100 * 1024 * 1024),
    )(x, Wq, K_ext, V_ext, Wo)


# device time: 41944 ns/iter; 2.8200x vs baseline; 1.8154x over previous
import jax
import jax.numpy as jnp
from jax import lax
from jax.experimental import pallas as pl
from jax.experimental.pallas import tpu as pltpu

N_DEV = 4
SQ = 256
SKV = 4096
HQ = 8
DH = 128
D = HQ * DH
QB = 64
NQB = 4
QROWS = HQ * QB
NCH = 2
CH = SKV // NCH
SCALE = 0.08838834764831843
BF = jnp.bfloat16


def kernel(x, Wq, K_ext, V_ext, Wo):
    def body(x_ref, wq_ref, k_ref, v_ref, wo_ref, out_ref,
             loc_o, loc_ml, recv_o, recv_ml, out_send, out_recv,
             k_vmem, v_vmem, wo_vmem, kv_sems, wo_sem,
             so_sems, ro_sems, sml_sems, rml_sems, sout_sems, rout_sems):
        my = lax.axis_index("i")
        peers = [lax.rem(my + d, N_DEV) for d in range(1, N_DEV)]

        def kv_copy(j, c):
            ref, dst = (k_ref, k_vmem) if j == 0 else (v_ref, v_vmem)
            return pltpu.make_async_copy(
                ref.at[0, pl.ds(c * CH, CH)],
                dst.at[0, pl.ds(c * CH, CH)],
                kv_sems.at[j, c])

        kv_copy(0, 0).start()
        kv_copy(1, 0).start()
        wo_copy = pltpu.make_async_copy(wo_ref, wo_vmem, wo_sem)
        wo_copy.start()

        barrier = pltpu.get_barrier_semaphore()
        for p in peers:
            pl.semaphore_signal(
                barrier, inc=1,
                device_id=(p,), device_id_type=pl.DeviceIdType.MESH,
            )
        pl.semaphore_wait(barrier, N_DEV - 1)

        xb = x_ref[0].astype(BF)
        q = jnp.dot(xb, wq_ref[...].astype(BF),
                    preferred_element_type=jnp.float32)
        qblks = [q[qb * QB:(qb + 1) * QB, :].astype(BF) for qb in range(NQB)]

        run_m = [[None] * HQ for _ in range(NQB)]
        run_l = [[None] * HQ for _ in range(NQB)]
        run_o = [[None] * HQ for _ in range(NQB)]

        for c in range(NCH):
            kv_copy(0, c).wait()
            kv_copy(1, c).wait()
            if c + 1 < NCH:
                kv_copy(0, c + 1).start()
                kv_copy(1, c + 1).start()
            kc = k_vmem[0, c * CH:(c + 1) * CH].reshape(CH, D)
            vc = v_vmem[0, c * CH:(c + 1) * CH].reshape(CH, D)
            for qb in range(NQB):
                ks = jnp.concatenate(
                    [kc[(qb + 4 * t) * 64:(qb + 4 * t + 1) * 64, :].astype(BF)
                     for t in range(CH // 256)], axis=0)
                vs = jnp.concatenate(
                    [vc[(qb + 4 * t) * 64:(qb + 4 * t + 1) * 64, :].astype(BF)
                     for t in range(CH // 256)], axis=0)
                for h in range(HQ):
                    hs = slice(h * DH, (h + 1) * DH)
                    s = lax.dot_general(
                        qblks[qb][:, hs], ks[:, hs], (((1,), (1,)), ((), ())),
                        preferred_element_type=jnp.float32) * SCALE
                    m_c = jnp.max(s, axis=1)
                    w = jnp.exp(s - m_c[:, None])
                    l_c = jnp.sum(w, axis=1)
                    o_c = jnp.dot(w.astype(BF), vs[:, hs],
                                  preferred_element_type=jnp.float32)
                    if c == 0:
                        run_m[qb][h] = m_c
                        run_l[qb][h] = l_c
                        run_o[qb][h] = o_c
                    else:
                        mn = jnp.maximum(run_m[qb][h], m_c)
                        a = jnp.exp(run_m[qb][h] - mn)
                        b = jnp.exp(m_c - mn)
                        run_o[qb][h] = (run_o[qb][h] * a[:, None]
                                        + o_c * b[:, None])
                        run_l[qb][h] = run_l[qb][h] * a + l_c * b
                        run_m[qb][h] = mn
                if c == NCH - 1:
                    for h in range(HQ):
                        r0 = qb * QROWS + h * QB
                        loc_o[r0:r0 + QB, :] = run_o[qb][h].astype(BF)
                        loc_ml[qb, h, :] = run_m[qb][h]
                        loc_ml[qb, HQ + h, :] = run_l[qb][h]

                    @pl.when(my != qb)
                    def _(qb=qb):
                        rdma_o = pltpu.make_async_remote_copy(
                            src_ref=loc_o.at[pl.ds(qb * QROWS, QROWS), :],
                            dst_ref=recv_o.at[my],
                            send_sem=so_sems.at[qb],
                            recv_sem=ro_sems.at[my],
                            device_id=(qb,),
                            device_id_type=pl.DeviceIdType.MESH,
                        )
                        rdma_ml = pltpu.make_async_remote_copy(
                            src_ref=loc_ml.at[qb],
                            dst_ref=recv_ml.at[my],
                            send_sem=sml_sems.at[qb],
                            recv_sem=rml_sems.at[my],
                            device_id=(qb,),
                            device_id_type=pl.DeviceIdType.MESH,
                        )
                        rdma_o.start()
                        rdma_ml.start()

        for p in peers:
            pltpu.make_async_remote_copy(
                src_ref=loc_o.at[pl.ds(0, QROWS), :],
                dst_ref=recv_o.at[p],
                send_sem=so_sems.at[0],
                recv_sem=ro_sems.at[p],
                device_id=(p,),
                device_id_type=pl.DeviceIdType.MESH,
            ).wait_recv()
            pltpu.make_async_remote_copy(
                src_ref=loc_ml.at[0],
                dst_ref=recv_ml.at[p],
                send_sem=sml_sems.at[0],
                recv_sem=rml_sems.at[p],
                device_id=(p,),
                device_id_type=pl.DeviceIdType.MESH,
            ).wait_recv()

        ctx_pieces = []
        for h in range(HQ):
            ms, ls, os_ = [], [], []
            ms.append(loc_ml[my, h, :])
            ls.append(loc_ml[my, HQ + h, :])
            os_.append(loc_o[pl.ds(my * QROWS + h * QB, QB), :])
            for p in peers:
                ms.append(recv_ml[p, h, :])
                ls.append(recv_ml[p, HQ + h, :])
                os_.append(recv_o[p, h * QB:(h + 1) * QB, :])
            gm = jnp.maximum(jnp.maximum(ms[0], ms[1]),
                             jnp.maximum(ms[2], ms[3]))
            num = jnp.zeros((QB, DH), jnp.float32)
            den = jnp.zeros((QB,), jnp.float32)
            for sl in range(N_DEV):
                sc = jnp.exp(ms[sl] - gm)
                den = den + sc * ls[sl]
                num = num + sc[:, None] * os_[sl].astype(jnp.float32)
            ctx_pieces.append(num / den[:, None])
        ctx_q = jnp.concatenate(ctx_pieces, axis=1)

        wo_copy.wait()
        out_q = jnp.dot(ctx_q.astype(BF), wo_vmem[...].astype(BF),
                        preferred_element_type=jnp.float32)
        out_ref[0, pl.ds(my * QB, QB), :] = out_q
        out_send[...] = out_q.astype(BF)

        out_rdmas = []
        for d, p in enumerate(peers):
            rdma = pltpu.make_async_remote_copy(
                src_ref=out_send,
                dst_ref=out_recv.at[my],
                send_sem=sout_sems.at[d],
                recv_sem=rout_sems.at[my],
                device_id=(p,),
                device_id_type=pl.DeviceIdType.MESH,
            )
            rdma.start()
            out_rdmas.append(rdma)
        for p in peers:
            pltpu.make_async_remote_copy(
                src_ref=out_send,
                dst_ref=out_recv.at[p],
                send_sem=sout_sems.at[0],
                recv_sem=rout_sems.at[p],
                device_id=(p,),
                device_id_type=pl.DeviceIdType.MESH,
            ).wait_recv()
            out_ref[0, pl.ds(p * QB, QB), :] = out_recv[p].astype(jnp.float32)

        for qb in range(NQB):
            @pl.when(my != qb)
            def _(qb=qb):
                pltpu.make_async_remote_copy(
                    src_ref=loc_o.at[pl.ds(qb * QROWS, QROWS), :],
                    dst_ref=recv_o.at[my],
                    send_sem=so_sems.at[qb],
                    recv_sem=ro_sems.at[my],
                    device_id=(qb,),
                    device_id_type=pl.DeviceIdType.MESH,
                ).wait_send()
                pltpu.make_async_remote_copy(
                    src_ref=loc_ml.at[qb],
                    dst_ref=recv_ml.at[my],
                    send_sem=sml_sems.at[qb],
                    recv_sem=rml_sems.at[my],
                    device_id=(qb,),
                    device_id_type=pl.DeviceIdType.MESH,
                ).wait_send()
        for rdma in out_rdmas:
            rdma.wait_send()

    return pl.pallas_call(
        body,
        out_shape=jax.ShapeDtypeStruct((1, SQ, D), jnp.float32),
        in_specs=[
            pl.BlockSpec(memory_space=pltpu.VMEM),
            pl.BlockSpec(memory_space=pltpu.VMEM),
            pl.BlockSpec(memory_space=pltpu.MemorySpace.HBM),
            pl.BlockSpec(memory_space=pltpu.MemorySpace.HBM),
            pl.BlockSpec(memory_space=pltpu.MemorySpace.HBM),
        ],
        out_specs=pl.BlockSpec(memory_space=pltpu.VMEM),
        scratch_shapes=[
            pltpu.VMEM((NQB * QROWS, DH), BF),
            pltpu.VMEM((NQB, 2 * HQ, QB), jnp.float32),
            pltpu.VMEM((N_DEV, QROWS, DH), BF),
            pltpu.VMEM((N_DEV, 2 * HQ, QB), jnp.float32),
            pltpu.VMEM((QB, D), BF),
            pltpu.VMEM((N_DEV, QB, D), BF),
            pltpu.VMEM((1, SKV, HQ, DH), jnp.float32),
            pltpu.VMEM((1, SKV, HQ, DH), jnp.float32),
            pltpu.VMEM((D, D), jnp.float32),
            pltpu.SemaphoreType.DMA((2, NCH)),
            pltpu.SemaphoreType.DMA,
            pltpu.SemaphoreType.DMA((NQB,)),
            pltpu.SemaphoreType.DMA((N_DEV,)),
            pltpu.SemaphoreType.DMA((NQB,)),
            pltpu.SemaphoreType.DMA((N_DEV,)),
            pltpu.SemaphoreType.DMA((N_DEV - 1,)),
            pltpu.SemaphoreType.DMA((N_DEV,)),
        ],
        compiler_params=pltpu.CompilerParams(
            collective_id=0, vmem_limit_bytes=100 * 1024 * 1024),
    )(x, Wq, K_ext, V_ext, Wo)
